# Initial kernel scaffold; baseline (speedup 1.0000x reference)
import jax
import jax.numpy as jnp
from jax.experimental import pallas as pl


def kernel(x, edge_index, W_l, W_r, b_l):
    raise NotImplementedError("write your pallas kernel here")


# speedup vs baseline: 12.6125x; 12.6125x over previous
import functools

import jax
import jax.numpy as jnp
from jax import lax
from jax.experimental import pallas as pl
from jax.experimental.pallas import tpu as pltpu
from jax.experimental.pallas import tpu_sc as plsc

NC = 2
NS = 16
NW = NC * NS
CH = 128

N = 10000
S = 24
F = 32
N_T = N + 16
N_A = 10240
ZROWS = N_A // NS


def _make_sc_agg(K):
    mesh = plsc.VectorSubcoreMesh(core_axis_name="c", subcore_axis_name="s")

    @functools.partial(
        pl.kernel,
        out_type=jax.ShapeDtypeStruct((NC * N_A, F), jnp.float32),
        mesh=mesh,
        compiler_params=pltpu.CompilerParams(use_tc_tiling_on_sc=False),
        scratch_types=[
            pltpu.VMEM((K, CH), jnp.int32),
            pltpu.VMEM((K, CH), jnp.int32),
            pltpu.VMEM((CH, F), jnp.float32),
            pltpu.VMEM((ZROWS, F), jnp.float32),
            pltpu.VMEM_SHARED((N_A, F), jnp.float32),
            pltpu.SemaphoreType.DMA,
        ],
    )
    def sc_agg(table_hbm, src_hbm, dst_hbm, out_hbm,
               src_v, dst_v, rows_v, stage_v, acc_sh, sem):
        cid = lax.axis_index("c")
        sid = lax.axis_index("s")
        wid = sid * NC + cid

        z16 = jnp.zeros((16,), jnp.float32)

        def _zrow(i, carry):
            stage_v[i, pl.ds(0, 16)] = z16
            stage_v[i, pl.ds(16, 16)] = z16
            return carry

        lax.fori_loop(0, ZROWS, _zrow, 0)
        pltpu.sync_copy(stage_v, acc_sh.at[pl.ds(sid * ZROWS, ZROWS)])

        pltpu.sync_copy(src_hbm.at[pl.ds(wid * K, K)], src_v)
        pltpu.sync_copy(dst_hbm.at[pl.ds(wid * K, K)], dst_v)

        plsc.subcore_barrier()

        def _edge_chunk(j, carry):
            pltpu.async_copy(table_hbm.at[src_v.at[j]], rows_v, sem).wait()
            pltpu.sync_copy(rows_v, acc_sh.at[dst_v.at[j]], add=True)
            return carry

        lax.fori_loop(0, K, _edge_chunk, 0)

        plsc.subcore_barrier()

        pltpu.sync_copy(acc_sh.at[pl.ds(sid * ZROWS, ZROWS)], stage_v)
        pltpu.sync_copy(stage_v, out_hbm.at[pl.ds(cid * N_A + sid * ZROWS, ZROWS)])

    return sc_agg


def _fin_body(x_ref, parts_ref, wl_ref, wr_ref, bl_ref, out_ref, mean_scr):
    b = pl.program_id(0)

    @pl.when(b == 0)
    def _():
        comb = parts_ref[0] + parts_ref[1]
        cnt = comb[S:S + 1, :]
        mean_scr[...] = comb[:S, :] / jnp.maximum(cnt, 1.0)

    wl = wl_ref[0, 0]
    wr = wr_ref[0, 0]
    bl = bl_ref[0]
    scale = jnp.where(b == 0, wl, jnp.float32(0.0))
    out_ref[0] = x_ref[0] * wr + bl + mean_scr[...] * scale


def kernel(x, edge_index, W_l, W_r, b_l):
    B, S_, N_ = x.shape
    E = edge_index.shape[1]
    K = -(-E // (NW * CH))
    K = -(-K // 8) * 8

    E_pad = NW * K * CH

    x0t = jnp.transpose(x[0], (1, 0))
    table = jnp.concatenate(
        [x0t, jnp.ones((N_, 1), x.dtype), jnp.zeros((N_, F - S_ - 1), x.dtype)],
        axis=1)
    table = jnp.concatenate([table, jnp.zeros((N_T - N_, F), x.dtype)], axis=0)

    src = jnp.concatenate(
        [edge_index[0], jnp.full((E_pad - E,), N_, jnp.int32)]).reshape(NW * K, CH)
    dst = jnp.concatenate(
        [edge_index[1], jnp.zeros((E_pad - E,), jnp.int32)]).reshape(NW * K, CH)

    parts = _make_sc_agg(K)(table, src, dst)
    parts_t = jnp.transpose(parts.reshape(NC, N_A, F)[:, :N_, :], (0, 2, 1))

    out = pl.pallas_call(
        _fin_body,
        grid=(B,),
        in_specs=[
            pl.BlockSpec((1, S_, N_), lambda b: (b, 0, 0)),
            pl.BlockSpec((NC, F, N_), lambda b: (0, 0, 0)),
            pl.BlockSpec(memory_space=pltpu.SMEM),
            pl.BlockSpec(memory_space=pltpu.SMEM),
            pl.BlockSpec(memory_space=pltpu.SMEM),
        ],
        out_specs=pl.BlockSpec((1, S_, N_), lambda b: (b, 0, 0)),
        out_shape=jax.ShapeDtypeStruct((B, S_, N_), jnp.float32),
        scratch_shapes=[pltpu.VMEM((S_, N_), jnp.float32)],
    )(x, parts_t, W_l, W_r, b_l)
    return out


# speedup vs baseline: 17.3535x; 1.3759x over previous
import functools

import jax
import jax.numpy as jnp
from jax import lax
from jax.experimental import pallas as pl
from jax.experimental.pallas import tpu as pltpu
from jax.experimental.pallas import tpu_sc as plsc

NC = 2
NS = 16
NW = NC * NS
CH = 125

N = 10000
S = 24
F = 32
N_T = N + 16
N_A = 10240
ZROWS = N_A // NS


def _make_sc_agg(K):
    mesh = plsc.VectorSubcoreMesh(core_axis_name="c", subcore_axis_name="s")

    @functools.partial(
        pl.kernel,
        out_type=jax.ShapeDtypeStruct((NC * N_A, F), jnp.float32),
        mesh=mesh,
        compiler_params=pltpu.CompilerParams(use_tc_tiling_on_sc=False),
        scratch_types=[
            pltpu.VMEM((K, CH), jnp.int32),
            pltpu.VMEM((K, CH), jnp.int32),
            pltpu.VMEM((CH, F), jnp.float32),
            pltpu.VMEM((ZROWS, F), jnp.float32),
            pltpu.VMEM_SHARED((N_A, F), jnp.float32),
            pltpu.SemaphoreType.DMA,
        ],
    )
    def sc_agg(table_hbm, src_hbm, dst_hbm, out_hbm,
               src_v, dst_v, rows_v, stage_v, acc_sh, sem):
        cid = lax.axis_index("c")
        sid = lax.axis_index("s")
        wid = sid * NC + cid

        z16 = jnp.zeros((16,), jnp.float32)

        def _zrow(i, carry):
            stage_v[i, pl.ds(0, 16)] = z16
            stage_v[i, pl.ds(16, 16)] = z16
            return carry

        lax.fori_loop(0, ZROWS, _zrow, 0)
        pltpu.sync_copy(stage_v, acc_sh.at[pl.ds(sid * ZROWS, ZROWS)])

        pltpu.sync_copy(src_hbm.at[pl.ds(wid * K, K)], src_v)
        pltpu.sync_copy(dst_hbm.at[pl.ds(wid * K, K)], dst_v)

        plsc.subcore_barrier()

        def _edge_chunk(j, carry):
            pltpu.async_copy(table_hbm.at[src_v.at[j]], rows_v, sem).wait()
            pltpu.sync_copy(rows_v, acc_sh.at[dst_v.at[j]], add=True)
            return carry

        lax.fori_loop(0, K, _edge_chunk, 0)

        plsc.subcore_barrier()

        pltpu.sync_copy(acc_sh.at[pl.ds(sid * ZROWS, ZROWS)], stage_v)
        pltpu.sync_copy(stage_v, out_hbm.at[pl.ds(cid * N_A + sid * ZROWS, ZROWS)])

    return sc_agg


def _fin_body(x_ref, parts_ref, wl_ref, wr_ref, bl_ref, out_ref, mean_scr):
    b = pl.program_id(0)

    @pl.when(b == 0)
    def _():
        comb = parts_ref[0] + parts_ref[1]
        cnt = comb[S:S + 1, :]
        mean_scr[...] = comb[:S, :] / jnp.maximum(cnt, 1.0)

    wl = wl_ref[0, 0]
    wr = wr_ref[0, 0]
    bl = bl_ref[0]
    scale = jnp.where(b == 0, wl, jnp.float32(0.0))
    out_ref[0] = x_ref[0] * wr + bl + mean_scr[...] * scale


def kernel(x, edge_index, W_l, W_r, b_l):
    B, S_, N_ = x.shape
    E = edge_index.shape[1]
    assert E % (NW * CH) == 0
    K = E // (NW * CH)

    x0t = jnp.transpose(x[0], (1, 0))
    table = jnp.concatenate(
        [x0t, jnp.ones((N_, 1), x.dtype), jnp.zeros((N_, F - S_ - 1), x.dtype)],
        axis=1)
    table = jnp.concatenate([table, jnp.zeros((N_T - N_, F), x.dtype)], axis=0)

    src = edge_index[0].reshape(NW * K, CH)
    dst = edge_index[1].reshape(NW * K, CH)

    parts = _make_sc_agg(K)(table, src, dst)
    parts_t = jnp.transpose(parts.reshape(NC, N_A, F)[:, :N_, :], (0, 2, 1))

    out = pl.pallas_call(
        _fin_body,
        grid=(B,),
        in_specs=[
            pl.BlockSpec((1, S_, N_), lambda b: (b, 0, 0)),
            pl.BlockSpec((NC, F, N_), lambda b: (0, 0, 0)),
            pl.BlockSpec(memory_space=pltpu.SMEM),
            pl.BlockSpec(memory_space=pltpu.SMEM),
            pl.BlockSpec(memory_space=pltpu.SMEM),
        ],
        out_specs=pl.BlockSpec((1, S_, N_), lambda b: (b, 0, 0)),
        out_shape=jax.ShapeDtypeStruct((B, S_, N_), jnp.float32),
        scratch_shapes=[pltpu.VMEM((S_, N_), jnp.float32)],
    )(x, parts_t, W_l, W_r, b_l)
    return out


# speedup vs baseline: 24.1437x; 1.3913x over previous
import functools

import jax
import jax.numpy as jnp
from jax import lax
from jax.experimental import pallas as pl
from jax.experimental.pallas import tpu as pltpu
from jax.experimental.pallas import tpu_sc as plsc

NC = 2
NS = 16
NW = NC * NS
CH = 125

N = 10000
S = 24
F = S + 1
N_T = N + 16
N_A = 10240
ZROWS = N_A // NS

_HI = jax.lax.Precision.HIGHEST


def _make_sc_agg(K):
    mesh = plsc.VectorSubcoreMesh(core_axis_name="c", subcore_axis_name="s")

    @functools.partial(
        pl.kernel,
        out_type=jax.ShapeDtypeStruct((NC * N_A, F), jnp.float32),
        mesh=mesh,
        compiler_params=pltpu.CompilerParams(use_tc_tiling_on_sc=False),
        scratch_types=[
            pltpu.VMEM((K, CH), jnp.int32),
            pltpu.VMEM((K, CH), jnp.int32),
            pltpu.VMEM((CH, F), jnp.float32),
            pltpu.VMEM((CH, F), jnp.float32),
            pltpu.VMEM((ZROWS, F), jnp.float32),
            pltpu.VMEM_SHARED((N_A, F), jnp.float32),
            pltpu.SemaphoreType.DMA,
            pltpu.SemaphoreType.DMA,
        ],
    )
    def sc_agg(edges_hbm, table_hbm, out_hbm,
               src_v, dst_v, rows_a, rows_b, stage_v, acc_sh, sem_a, sem_b):
        cid = lax.axis_index("c")
        sid = lax.axis_index("s")
        wid = sid * NC + cid

        z16 = jnp.zeros((16,), jnp.float32)

        def _zrow(i, carry):
            stage_v[i, pl.ds(0, 16)] = z16
            stage_v[i, pl.ds(F - 16, 16)] = z16
            return carry

        lax.fori_loop(0, ZROWS, _zrow, 0)
        pltpu.sync_copy(stage_v, acc_sh.at[pl.ds(sid * ZROWS, ZROWS)])

        pltpu.sync_copy(edges_hbm.at[0, pl.ds(wid * K, K)], src_v)
        pltpu.sync_copy(edges_hbm.at[1, pl.ds(wid * K, K)], dst_v)

        plsc.subcore_barrier()

        def _gather(j, buf, sem):
            pltpu.async_copy(table_hbm.at[src_v.at[j]], buf, sem)

        def _wait_scatter(j, buf, sem):
            pltpu.make_async_copy(table_hbm.at[src_v.at[j]], buf, sem).wait()
            pltpu.sync_copy(buf, acc_sh.at[dst_v.at[j]], add=True)

        _gather(0, rows_a, sem_a)
        _gather(1, rows_b, sem_b)

        def _pair(jj, carry):
            j2 = 2 * jj
            _wait_scatter(j2, rows_a, sem_a)
            _gather(j2 + 2, rows_a, sem_a)
            _wait_scatter(j2 + 1, rows_b, sem_b)
            _gather(j2 + 3, rows_b, sem_b)
            return carry

        lax.fori_loop(0, K // 2 - 1, _pair, 0)
        _wait_scatter(K - 2, rows_a, sem_a)
        _wait_scatter(K - 1, rows_b, sem_b)

        plsc.subcore_barrier()

        pltpu.sync_copy(acc_sh.at[pl.ds(sid * ZROWS, ZROWS)], stage_v)
        pltpu.sync_copy(stage_v, out_hbm.at[pl.ds(cid * N_A + sid * ZROWS, ZROWS)])

    return sc_agg


def _tbl_body(x_ref, tbl_ref):
    x0 = x_ref[0]
    eye = jnp.eye(S, dtype=jnp.float32)
    xt = lax.dot_general(x0, eye, (((0,), (0,)), ((), ())), precision=_HI)
    tbl = jnp.concatenate([xt, jnp.ones((N, 1), jnp.float32)], axis=1)
    tbl = jnp.concatenate(
        [tbl, jnp.zeros((N_T - N, F), jnp.float32)], axis=0)
    tbl_ref[...] = tbl


def _fin_body(x_ref, parts_ref, wl_ref, wr_ref, bl_ref, out_ref, mean_scr):
    b = pl.program_id(0)

    @pl.when(b == 0)
    def _():
        p = parts_ref[...]
        comb = p[0:N, :] + p[N_A:N_A + N, :]
        mean_nf = comb[:, 0:S] / jnp.maximum(comb[:, S:S + 1], 1.0)
        eye = jnp.eye(S, dtype=jnp.float32)
        mean_scr[...] = lax.dot_general(
            eye, mean_nf, (((1,), (1,)), ((), ())), precision=_HI)

    wl = wl_ref[0, 0]
    wr = wr_ref[0, 0]
    bl = bl_ref[0]
    scale = jnp.where(b == 0, wl, jnp.float32(0.0))
    out_ref[0] = x_ref[0] * wr + bl + mean_scr[...] * scale


def kernel(x, edge_index, W_l, W_r, b_l):
    B, S_, N_ = x.shape
    E = edge_index.shape[1]
    assert E % (NW * CH) == 0
    K = E // (NW * CH)

    table = pl.pallas_call(
        _tbl_body,
        grid=(1,),
        in_specs=[pl.BlockSpec((1, S_, N_), lambda i: (0, 0, 0))],
        out_specs=pl.BlockSpec((N_T, F), lambda i: (0, 0)),
        out_shape=jax.ShapeDtypeStruct((N_T, F), jnp.float32),
    )(x)

    edges = edge_index.reshape(2, NW * K, CH)
    parts = _make_sc_agg(K)(edges, table)

    out = pl.pallas_call(
        _fin_body,
        grid=(B,),
        in_specs=[
            pl.BlockSpec((1, S_, N_), lambda b: (b, 0, 0)),
            pl.BlockSpec((NC * N_A, F), lambda b: (0, 0)),
            pl.BlockSpec(memory_space=pltpu.SMEM),
            pl.BlockSpec(memory_space=pltpu.SMEM),
            pl.BlockSpec(memory_space=pltpu.SMEM),
        ],
        out_specs=pl.BlockSpec((1, S_, N_), lambda b: (b, 0, 0)),
        out_shape=jax.ShapeDtypeStruct((B, S_, N_), jnp.float32),
        scratch_shapes=[pltpu.VMEM((S_, N_), jnp.float32)],
    )(x, parts, W_l, W_r, b_l)
    return out


# speedup vs baseline: 24.9303x; 1.0326x over previous
import functools

import jax
import jax.numpy as jnp
from jax import lax
from jax.experimental import pallas as pl
from jax.experimental.pallas import tpu as pltpu
from jax.experimental.pallas import tpu_sc as plsc

NC = 2
NS = 16
NW = NC * NS
CH = 125

N = 10000
S = 24
F = 32
N_T = N + 16
N_A = 10240
ZROWS = N_A // NS

_HI = jax.lax.Precision.HIGHEST


def _make_sc_agg(K):
    mesh = plsc.VectorSubcoreMesh(core_axis_name="c", subcore_axis_name="s")

    @functools.partial(
        pl.kernel,
        out_type=jax.ShapeDtypeStruct((NC * N_A, F), jnp.float32),
        mesh=mesh,
        compiler_params=pltpu.CompilerParams(use_tc_tiling_on_sc=False),
        scratch_types=[
            pltpu.VMEM((K, CH), jnp.int32),
            pltpu.VMEM((K, CH), jnp.int32),
            pltpu.VMEM((CH, F), jnp.float32),
            pltpu.VMEM((CH, F), jnp.float32),
            pltpu.VMEM((ZROWS, F), jnp.float32),
            pltpu.VMEM_SHARED((N_A, F), jnp.float32),
            pltpu.SemaphoreType.DMA,
            pltpu.SemaphoreType.DMA,
        ],
    )
    def sc_agg(edges_hbm, table_hbm, out_hbm,
               src_v, dst_v, rows_a, rows_b, stage_v, acc_sh, sem_a, sem_b):
        cid = lax.axis_index("c")
        sid = lax.axis_index("s")
        wid = sid * NC + cid

        z16 = jnp.zeros((16,), jnp.float32)

        def _zrow(i, carry):
            stage_v[i, pl.ds(0, 16)] = z16
            stage_v[i, pl.ds(F - 16, 16)] = z16
            return carry

        lax.fori_loop(0, ZROWS, _zrow, 0)
        pltpu.sync_copy(stage_v, acc_sh.at[pl.ds(sid * ZROWS, ZROWS)])

        pltpu.sync_copy(edges_hbm.at[0, pl.ds(wid * K, K)], src_v)
        pltpu.sync_copy(edges_hbm.at[1, pl.ds(wid * K, K)], dst_v)

        plsc.subcore_barrier()

        def _gather(j, buf, sem):
            pltpu.async_copy(table_hbm.at[src_v.at[j]], buf, sem)

        def _wait_scatter(j, buf, sem):
            pltpu.make_async_copy(table_hbm.at[src_v.at[j]], buf, sem).wait()
            pltpu.sync_copy(buf, acc_sh.at[dst_v.at[j]], add=True)

        _gather(0, rows_a, sem_a)
        _gather(1, rows_b, sem_b)

        def _pair(jj, carry):
            j2 = 2 * jj
            _wait_scatter(j2, rows_a, sem_a)
            _gather(j2 + 2, rows_a, sem_a)
            _wait_scatter(j2 + 1, rows_b, sem_b)
            _gather(j2 + 3, rows_b, sem_b)
            return carry

        lax.fori_loop(0, K // 2 - 1, _pair, 0)
        _wait_scatter(K - 2, rows_a, sem_a)
        _wait_scatter(K - 1, rows_b, sem_b)

        plsc.subcore_barrier()

        pltpu.sync_copy(acc_sh.at[pl.ds(sid * ZROWS, ZROWS)], stage_v)
        pltpu.sync_copy(stage_v, out_hbm.at[pl.ds(cid * N_A + sid * ZROWS, ZROWS)])

    return sc_agg


def _tbl_body(x_ref, tbl_ref):
    x0 = x_ref[0]
    eye = jnp.eye(S, dtype=jnp.float32)
    xt = lax.dot_general(x0, eye, (((0,), (0,)), ((), ())), precision=_HI)
    tbl = jnp.concatenate(
        [xt, jnp.ones((N, 1), jnp.float32),
         jnp.zeros((N, F - S - 1), jnp.float32)], axis=1)
    tbl = jnp.concatenate(
        [tbl, jnp.zeros((N_T - N, F), jnp.float32)], axis=0)
    tbl_ref[...] = tbl


def _fin_body(x_ref, parts_ref, wl_ref, wr_ref, bl_ref, out_ref, mean_scr):
    b = pl.program_id(0)

    @pl.when(b == 0)
    def _():
        p = parts_ref[...]
        comb = p[0:N, :] + p[N_A:N_A + N, :]
        mean_nf = comb[:, 0:S] / jnp.maximum(comb[:, S:S + 1], 1.0)
        eye = jnp.eye(S, dtype=jnp.float32)
        mean_scr[...] = lax.dot_general(
            eye, mean_nf, (((1,), (1,)), ((), ())), precision=_HI)

    wl = wl_ref[0, 0]
    wr = wr_ref[0, 0]
    bl = bl_ref[0]
    scale = jnp.where(b == 0, wl, jnp.float32(0.0))
    out_ref[0] = x_ref[0] * wr + bl + mean_scr[...] * scale


def kernel(x, edge_index, W_l, W_r, b_l):
    B, S_, N_ = x.shape
    E = edge_index.shape[1]
    assert E % (NW * CH) == 0
    K = E // (NW * CH)

    table = pl.pallas_call(
        _tbl_body,
        grid=(1,),
        in_specs=[pl.BlockSpec((1, S_, N_), lambda i: (0, 0, 0))],
        out_specs=pl.BlockSpec((N_T, F), lambda i: (0, 0)),
        out_shape=jax.ShapeDtypeStruct((N_T, F), jnp.float32),
    )(x)

    edges = edge_index.reshape(2, NW * K, CH)
    parts = _make_sc_agg(K)(edges, table)

    out = pl.pallas_call(
        _fin_body,
        grid=(B,),
        in_specs=[
            pl.BlockSpec((1, S_, N_), lambda b: (b, 0, 0)),
            pl.BlockSpec((NC * N_A, F), lambda b: (0, 0)),
            pl.BlockSpec(memory_space=pltpu.SMEM),
            pl.BlockSpec(memory_space=pltpu.SMEM),
            pl.BlockSpec(memory_space=pltpu.SMEM),
        ],
        out_specs=pl.BlockSpec((1, S_, N_), lambda b: (b, 0, 0)),
        out_shape=jax.ShapeDtypeStruct((B, S_, N_), jnp.float32),
        scratch_shapes=[pltpu.VMEM((S_, N_), jnp.float32)],
    )(x, parts, W_l, W_r, b_l)
    return out


# speedup vs baseline: 27.9013x; 1.1192x over previous
import functools

import jax
import jax.numpy as jnp
from jax import lax
from jax.experimental import pallas as pl
from jax.experimental.pallas import tpu as pltpu
from jax.experimental.pallas import tpu_sc as plsc

NC = 2
NS = 16
NW = NC * NS
CH = 125

N = 10000
S = 24
F = 32
N_T = N + 16
N_A = 10240
ZROWS = N_A // NS

_HI = jax.lax.Precision.HIGHEST


def _make_sc_agg(K):
    mesh = plsc.VectorSubcoreMesh(core_axis_name="c", subcore_axis_name="s")

    @functools.partial(
        pl.kernel,
        out_type=jax.ShapeDtypeStruct((NC * N_A, F), jnp.float32),
        mesh=mesh,
        compiler_params=pltpu.CompilerParams(use_tc_tiling_on_sc=False),
        scratch_types=[
            pltpu.VMEM((K, CH), jnp.int32),
            pltpu.VMEM((K, CH), jnp.int32),
            pltpu.VMEM((CH, F), jnp.float32),
            pltpu.VMEM((CH, F), jnp.float32),
            pltpu.VMEM((CH, F), jnp.float32),
            pltpu.VMEM((CH, F), jnp.float32),
            pltpu.VMEM((ZROWS, F), jnp.float32),
            pltpu.VMEM_SHARED((N_A, F), jnp.float32),
            pltpu.SemaphoreType.DMA,
            pltpu.SemaphoreType.DMA,
            pltpu.SemaphoreType.DMA,
            pltpu.SemaphoreType.DMA,
            pltpu.SemaphoreType.DMA,
            pltpu.SemaphoreType.DMA,
            pltpu.SemaphoreType.DMA,
            pltpu.SemaphoreType.DMA,
        ],
    )
    def sc_agg(edges_hbm, table_hbm, out_hbm,
               src_v, dst_v, rows_a, rows_b, rows_c, rows_d, stage_v, acc_sh,
               ga, gb, gc, gd, sa, sb, sc, sd):
        cid = lax.axis_index("c")
        sid = lax.axis_index("s")
        wid = sid * NC + cid

        z16 = jnp.zeros((16,), jnp.float32)

        def _zrow(i, carry):
            stage_v[i, pl.ds(0, 16)] = z16
            stage_v[i, pl.ds(F - 16, 16)] = z16
            return carry

        lax.fori_loop(0, ZROWS, _zrow, 0)
        pltpu.sync_copy(stage_v, acc_sh.at[pl.ds(sid * ZROWS, ZROWS)])

        pltpu.sync_copy(edges_hbm.at[0, pl.ds(wid * K, K)], src_v)
        pltpu.sync_copy(edges_hbm.at[1, pl.ds(wid * K, K)], dst_v)

        plsc.subcore_barrier()

        bufs = (rows_a, rows_b, rows_c, rows_d)
        gsems = (ga, gb, gc, gd)
        ssems = (sa, sb, sc, sd)

        def _gather(j, i):
            pltpu.async_copy(table_hbm.at[src_v.at[j]], bufs[i], gsems[i])

        def _wait_gather(j, i):
            pltpu.make_async_copy(
                table_hbm.at[src_v.at[j]], bufs[i], gsems[i]).wait()

        def _scatter(j, i):
            pltpu.async_copy(bufs[i], acc_sh.at[dst_v.at[j]], ssems[i],
                             add=True)

        def _wait_scatter(j, i):
            pltpu.make_async_copy(
                bufs[i], acc_sh.at[dst_v.at[j]], ssems[i]).wait()

        for i in range(4):
            _gather(i, i)

        def _quad(qq, carry):
            j = 4 * qq
            for i in range(4):
                _wait_gather(j + i, i)
                _scatter(j + i, i)
            for i in range(4):
                _wait_scatter(j + i, i)
                _gather(j + 4 + i, i)
            return carry

        lax.fori_loop(0, K // 4 - 1, _quad, 0)
        for i in range(4):
            _wait_gather(K - 4 + i, i)
            _scatter(K - 4 + i, i)
        for i in range(4):
            _wait_scatter(K - 4 + i, i)

        plsc.subcore_barrier()

        pltpu.sync_copy(acc_sh.at[pl.ds(sid * ZROWS, ZROWS)], stage_v)
        pltpu.sync_copy(stage_v, out_hbm.at[pl.ds(cid * N_A + sid * ZROWS, ZROWS)])

    return sc_agg


def _tbl_body(x_ref, tbl_ref):
    x0 = x_ref[0]
    eye = jnp.eye(S, dtype=jnp.float32)
    xt = lax.dot_general(x0, eye, (((0,), (0,)), ((), ())), precision=_HI)
    tbl = jnp.concatenate(
        [xt, jnp.ones((N, 1), jnp.float32),
         jnp.zeros((N, F - S - 1), jnp.float32)], axis=1)
    tbl = jnp.concatenate(
        [tbl, jnp.zeros((N_T - N, F), jnp.float32)], axis=0)
    tbl_ref[...] = tbl


def _fin_body(x_ref, parts_ref, wl_ref, wr_ref, bl_ref, out_ref, mean_scr):
    b = pl.program_id(0)

    @pl.when(b == 0)
    def _():
        p = parts_ref[...]
        comb = p[0:N, :] + p[N_A:N_A + N, :]
        mean_nf = comb[:, 0:S] / jnp.maximum(comb[:, S:S + 1], 1.0)
        eye = jnp.eye(S, dtype=jnp.float32)
        mean_scr[...] = lax.dot_general(
            eye, mean_nf, (((1,), (1,)), ((), ())), precision=_HI)

    wl = wl_ref[0, 0]
    wr = wr_ref[0, 0]
    bl = bl_ref[0]
    scale = jnp.where(b == 0, wl, jnp.float32(0.0))
    out_ref[0] = x_ref[0] * wr + bl + mean_scr[...] * scale


def kernel(x, edge_index, W_l, W_r, b_l):
    B, S_, N_ = x.shape
    E = edge_index.shape[1]
    assert E % (NW * CH) == 0
    K = E // (NW * CH)

    table = pl.pallas_call(
        _tbl_body,
        grid=(1,),
        in_specs=[pl.BlockSpec((1, S_, N_), lambda i: (0, 0, 0))],
        out_specs=pl.BlockSpec((N_T, F), lambda i: (0, 0)),
        out_shape=jax.ShapeDtypeStruct((N_T, F), jnp.float32),
    )(x)

    edges = edge_index.reshape(2, NW * K, CH)
    parts = _make_sc_agg(K)(edges, table)

    out = pl.pallas_call(
        _fin_body,
        grid=(B,),
        in_specs=[
            pl.BlockSpec((1, S_, N_), lambda b: (b, 0, 0)),
            pl.BlockSpec((NC * N_A, F), lambda b: (0, 0)),
            pl.BlockSpec(memory_space=pltpu.SMEM),
            pl.BlockSpec(memory_space=pltpu.SMEM),
            pl.BlockSpec(memory_space=pltpu.SMEM),
        ],
        out_specs=pl.BlockSpec((1, S_, N_), lambda b: (b, 0, 0)),
        out_shape=jax.ShapeDtypeStruct((B, S_, N_), jnp.float32),
        scratch_shapes=[pltpu.VMEM((S_, N_), jnp.float32)],
    )(x, parts, W_l, W_r, b_l)
    return out


# speedup vs baseline: 28.7576x; 1.0307x over previous
import functools

import jax
import jax.numpy as jnp
from jax import lax
from jax.experimental import pallas as pl
from jax.experimental.pallas import tpu as pltpu
from jax.experimental.pallas import tpu_sc as plsc

NC = 2
NS = 16
NW = NC * NS
CH = 125

N = 10000
S = 24
F = 32
N_T = N + 16
N_A = 10240
ZROWS = N_A // NS

_HI = jax.lax.Precision.HIGHEST


def _make_sc_agg(K):
    mesh = plsc.VectorSubcoreMesh(core_axis_name="c", subcore_axis_name="s")

    @functools.partial(
        pl.kernel,
        out_type=jax.ShapeDtypeStruct((NC * N_A, F), jnp.float32),
        mesh=mesh,
        compiler_params=pltpu.CompilerParams(use_tc_tiling_on_sc=False),
        scratch_types=[
            pltpu.VMEM((K, CH), jnp.int32),
            pltpu.VMEM((K, CH), jnp.int32),
            pltpu.VMEM((CH, F), jnp.float32),
            pltpu.VMEM((CH, F), jnp.float32),
            pltpu.VMEM((CH, F), jnp.float32),
            pltpu.VMEM((CH, F), jnp.float32),
            pltpu.VMEM((ZROWS, F), jnp.float32),
            pltpu.VMEM_SHARED((N_A, F), jnp.float32),
            pltpu.SemaphoreType.DMA,
            pltpu.SemaphoreType.DMA,
            pltpu.SemaphoreType.DMA,
            pltpu.SemaphoreType.DMA,
            pltpu.SemaphoreType.DMA,
            pltpu.SemaphoreType.DMA,
            pltpu.SemaphoreType.DMA,
            pltpu.SemaphoreType.DMA,
        ],
    )
    def sc_agg(edges_hbm, table_hbm, out_hbm,
               src_v, dst_v, rows_a, rows_b, rows_c, rows_d, stage_v, acc_sh,
               ga, gb, gc, gd, sa, sb, sc, sd):
        cid = lax.axis_index("c")
        sid = lax.axis_index("s")
        wid = sid * NC + cid

        z16 = jnp.zeros((16,), jnp.float32)

        def _zrow(i, carry):
            stage_v[i, pl.ds(0, 16)] = z16
            stage_v[i, pl.ds(F - 16, 16)] = z16
            return carry

        lax.fori_loop(0, ZROWS, _zrow, 0)
        pltpu.sync_copy(stage_v, acc_sh.at[pl.ds(sid * ZROWS, ZROWS)])

        pltpu.sync_copy(edges_hbm.at[0, pl.ds(wid * K, K)], src_v)
        pltpu.sync_copy(edges_hbm.at[1, pl.ds(wid * K, K)], dst_v)

        plsc.subcore_barrier()

        bufs = (rows_a, rows_b, rows_c, rows_d)
        gsems = (ga, gb, gc, gd)
        ssems = (sa, sb, sc, sd)

        def _gather(j, i):
            pltpu.async_copy(table_hbm.at[src_v.at[j]], bufs[i], gsems[i])

        def _wait_gather(j, i):
            pltpu.make_async_copy(
                table_hbm.at[src_v.at[j]], bufs[i], gsems[i]).wait()

        def _scatter(j, i):
            pltpu.async_copy(bufs[i], acc_sh.at[dst_v.at[j]], ssems[i],
                             add=True)

        def _wait_scatter(j, i):
            pltpu.make_async_copy(
                bufs[i], acc_sh.at[dst_v.at[j]], ssems[i]).wait()

        for i in range(4):
            _gather(i, i)

        def _quad(qq, carry):
            j = 4 * qq
            for i in range(4):
                _wait_gather(j + i, i)
                _scatter(j + i, i)
                _wait_scatter(j + i, i)
                _gather(j + 4 + i, i)
            return carry

        lax.fori_loop(0, K // 4 - 1, _quad, 0)
        for i in range(4):
            _wait_gather(K - 4 + i, i)
            _scatter(K - 4 + i, i)
            _wait_scatter(K - 4 + i, i)

        plsc.subcore_barrier()

        pltpu.sync_copy(acc_sh.at[pl.ds(sid * ZROWS, ZROWS)], stage_v)
        pltpu.sync_copy(stage_v, out_hbm.at[pl.ds(cid * N_A + sid * ZROWS, ZROWS)])

    return sc_agg


def _tbl_body(x_ref, tbl_ref):
    x0 = x_ref[0]
    eye = jnp.eye(S, dtype=jnp.float32)
    xt = lax.dot_general(x0, eye, (((0,), (0,)), ((), ())), precision=_HI)
    tbl = jnp.concatenate(
        [xt, jnp.ones((N, 1), jnp.float32),
         jnp.zeros((N, F - S - 1), jnp.float32)], axis=1)
    tbl = jnp.concatenate(
        [tbl, jnp.zeros((N_T - N, F), jnp.float32)], axis=0)
    tbl_ref[...] = tbl


def _fin_body(x_ref, parts_ref, wl_ref, wr_ref, bl_ref, out_ref, mean_scr):
    b = pl.program_id(0)

    @pl.when(b == 0)
    def _():
        p = parts_ref[...]
        comb = p[0:N, :] + p[N_A:N_A + N, :]
        mean_nf = comb[:, 0:S] / jnp.maximum(comb[:, S:S + 1], 1.0)
        eye = jnp.eye(S, dtype=jnp.float32)
        mean_scr[...] = lax.dot_general(
            eye, mean_nf, (((1,), (1,)), ((), ())), precision=_HI)

    wl = wl_ref[0, 0]
    wr = wr_ref[0, 0]
    bl = bl_ref[0]
    scale = jnp.where(b == 0, wl, jnp.float32(0.0))
    out_ref[0] = x_ref[0] * wr + bl + mean_scr[...] * scale


def kernel(x, edge_index, W_l, W_r, b_l):
    B, S_, N_ = x.shape
    E = edge_index.shape[1]
    assert E % (NW * CH) == 0
    K = E // (NW * CH)

    table = pl.pallas_call(
        _tbl_body,
        grid=(1,),
        in_specs=[pl.BlockSpec((1, S_, N_), lambda i: (0, 0, 0))],
        out_specs=pl.BlockSpec((N_T, F), lambda i: (0, 0)),
        out_shape=jax.ShapeDtypeStruct((N_T, F), jnp.float32),
    )(x)

    edges = edge_index.reshape(2, NW * K, CH)
    parts = _make_sc_agg(K)(edges, table)

    out = pl.pallas_call(
        _fin_body,
        grid=(B,),
        in_specs=[
            pl.BlockSpec((1, S_, N_), lambda b: (b, 0, 0)),
            pl.BlockSpec((NC * N_A, F), lambda b: (0, 0)),
            pl.BlockSpec(memory_space=pltpu.SMEM),
            pl.BlockSpec(memory_space=pltpu.SMEM),
            pl.BlockSpec(memory_space=pltpu.SMEM),
        ],
        out_specs=pl.BlockSpec((1, S_, N_), lambda b: (b, 0, 0)),
        out_shape=jax.ShapeDtypeStruct((B, S_, N_), jnp.float32),
        scratch_shapes=[pltpu.VMEM((S_, N_), jnp.float32)],
    )(x, parts, W_l, W_r, b_l)
    return out


# speedup vs baseline: 31.5534x; 1.0972x over previous
import functools

import jax
import jax.numpy as jnp
from jax import lax
from jax.experimental import pallas as pl
from jax.experimental.pallas import tpu as pltpu
from jax.experimental.pallas import tpu_sc as plsc

NC = 2
NS = 16
NW = NC * NS
CH = 125

N = 10000
S = 24
F = 32
N_T = N + 16
N_A = 10240
ZROWS = N_A // NS

_HI = jax.lax.Precision.HIGHEST


def _make_sc_agg(K):
    mesh = plsc.VectorSubcoreMesh(core_axis_name="c", subcore_axis_name="s")

    @functools.partial(
        pl.kernel,
        out_type=jax.ShapeDtypeStruct((NC * N_A, F), jnp.float32),
        mesh=mesh,
        compiler_params=pltpu.CompilerParams(use_tc_tiling_on_sc=False),
        scratch_types=[
            pltpu.VMEM((K, CH), jnp.int32),
            pltpu.VMEM((K, CH), jnp.int32),
            pltpu.VMEM((CH, F), jnp.float32),
            pltpu.VMEM((CH, F), jnp.float32),
            pltpu.VMEM((CH, F), jnp.float32),
            pltpu.VMEM((CH, F), jnp.float32),
            pltpu.VMEM((ZROWS, F), jnp.float32),
            pltpu.VMEM_SHARED((N_A, F), jnp.float32),
            pltpu.SemaphoreType.DMA,
            pltpu.SemaphoreType.DMA,
            pltpu.SemaphoreType.DMA,
            pltpu.SemaphoreType.DMA,
            pltpu.SemaphoreType.DMA,
            pltpu.SemaphoreType.DMA,
            pltpu.SemaphoreType.DMA,
            pltpu.SemaphoreType.DMA,
        ],
    )
    def sc_agg(edges_hbm, table_hbm, out_hbm,
               src_v, dst_v, rows_a, rows_b, rows_c, rows_d, stage_v, acc_sh,
               ga, gb, gc, gd, sa, sb, sc, sd):
        cid = lax.axis_index("c")
        sid = lax.axis_index("s")
        wid = sid * NC + cid

        z16 = jnp.zeros((16,), jnp.float32)

        def _zrow(i, carry):
            stage_v[i, pl.ds(0, 16)] = z16
            stage_v[i, pl.ds(F - 16, 16)] = z16
            return carry

        lax.fori_loop(0, ZROWS, _zrow, 0)
        pltpu.sync_copy(stage_v, acc_sh.at[pl.ds(sid * ZROWS, ZROWS)])

        pltpu.sync_copy(edges_hbm.at[0, pl.ds(wid * K, K)], src_v)
        pltpu.sync_copy(edges_hbm.at[1, pl.ds(wid * K, K)], dst_v)

        plsc.subcore_barrier()

        bufs = (rows_a, rows_b, rows_c, rows_d)
        gsems = (ga, gb, gc, gd)
        ssems = (sa, sb, sc, sd)

        def _gather(j, i):
            pltpu.async_copy(table_hbm.at[src_v.at[j]], bufs[i], gsems[i])

        def _wait_gather(j, i):
            pltpu.make_async_copy(
                table_hbm.at[src_v.at[j]], bufs[i], gsems[i]).wait()

        def _scatter(j, i):
            pltpu.async_copy(bufs[i], acc_sh.at[dst_v.at[j]], ssems[i],
                             add=True)

        def _wait_scatter(j, i):
            pltpu.make_async_copy(
                bufs[i], acc_sh.at[dst_v.at[j]], ssems[i]).wait()

        for i in range(4):
            _gather(i, i)

        def _quad(qq, carry):
            j = 4 * qq
            for i in range(4):
                _wait_gather(j + i, i)
                _scatter(j + i, i)
                _wait_scatter(j + i, i)
                _gather(j + 4 + i, i)
            return carry

        lax.fori_loop(0, K // 4 - 1, _quad, 0)
        for i in range(4):
            _wait_gather(K - 4 + i, i)
            _scatter(K - 4 + i, i)
            _wait_scatter(K - 4 + i, i)

        plsc.subcore_barrier()

        pltpu.sync_copy(acc_sh.at[pl.ds(sid * ZROWS, ZROWS)], stage_v)
        pltpu.sync_copy(stage_v, out_hbm.at[pl.ds(cid * N_A + sid * ZROWS, ZROWS)])

    return sc_agg


def _tbl_body(x_ref, tbl_ref):
    x0 = x_ref[0]
    eye = jnp.eye(S, dtype=jnp.float32)
    xt = lax.dot_general(x0, eye, (((0,), (0,)), ((), ())), precision=_HI)
    tbl = jnp.concatenate(
        [xt, jnp.ones((N, 1), jnp.float32),
         jnp.zeros((N, F - S - 1), jnp.float32)], axis=1)
    tbl = jnp.concatenate(
        [tbl, jnp.zeros((N_T - N, F), jnp.float32)], axis=0)
    tbl_ref[...] = tbl


def _fin_rest_body(x_ref, wr_ref, bl_ref, out_ref):
    out_ref[0] = x_ref[0] * wr_ref[0, 0] + bl_ref[0]


def _fin_b0_body(rest_ref, parts_ref, wl_ref, out_ref):
    p = parts_ref[...]
    comb = p[0:N, :] + p[N_A:N_A + N, :]
    mean_nf = comb[:, 0:S] / jnp.maximum(comb[:, S:S + 1], 1.0)
    eye = jnp.eye(S, dtype=jnp.float32)
    mean_t = lax.dot_general(
        eye, mean_nf, (((1,), (1,)), ((), ())), precision=_HI)
    out_ref[0] = rest_ref[0] + wl_ref[0, 0] * mean_t


def kernel(x, edge_index, W_l, W_r, b_l):
    B, S_, N_ = x.shape
    E = edge_index.shape[1]
    assert E % (NW * CH) == 0
    K = E // (NW * CH)

    table = pl.pallas_call(
        _tbl_body,
        grid=(1,),
        in_specs=[pl.BlockSpec((1, S_, N_), lambda i: (0, 0, 0))],
        out_specs=pl.BlockSpec((N_T, F), lambda i: (0, 0)),
        out_shape=jax.ShapeDtypeStruct((N_T, F), jnp.float32),
    )(x)

    edges = edge_index.reshape(2, NW * K, CH)
    parts = _make_sc_agg(K)(edges, table)

    rest = pl.pallas_call(
        _fin_rest_body,
        grid=(B,),
        in_specs=[
            pl.BlockSpec((1, S_, N_), lambda b: (b, 0, 0)),
            pl.BlockSpec(memory_space=pltpu.SMEM),
            pl.BlockSpec(memory_space=pltpu.SMEM),
        ],
        out_specs=pl.BlockSpec((1, S_, N_), lambda b: (b, 0, 0)),
        out_shape=jax.ShapeDtypeStruct((B, S_, N_), jnp.float32),
    )(x, W_r, b_l)

    out = pl.pallas_call(
        _fin_b0_body,
        grid=(1,),
        in_specs=[
            pl.BlockSpec((1, S_, N_), lambda i: (0, 0, 0)),
            pl.BlockSpec((NC * N_A, F), lambda i: (0, 0)),
            pl.BlockSpec(memory_space=pltpu.SMEM),
        ],
        out_specs=pl.BlockSpec((1, S_, N_), lambda i: (0, 0, 0)),
        out_shape=jax.ShapeDtypeStruct((B, S_, N_), jnp.float32),
        input_output_aliases={0: 0},
    )(rest, parts, W_l)
    return out
